# baseline (speedup 1.0000x reference)
import functools

import jax
import jax.numpy as jnp
from jax import lax
from jax.experimental import pallas as pl
from jax.experimental.pallas import tpu as pltpu
from jax.experimental.pallas import tpu_sc as plsc

_D = 64
_NW = 32
_NBUF = 4
_LOOK = 2
_EPS = 1e-5


def _rsqrt(x):
    i = plsc.bitcast(x, jnp.int32)
    i = jnp.int32(0x5F3759DF) - lax.shift_right_arithmetic(i, 1)
    y = plsc.bitcast(i, jnp.float32)
    h = x * 0.5
    for _ in range(3):
        y = y * (1.5 - h * y * y)
    return y


def _make_kernel(batch, hist):
    mesh = plsc.VectorSubcoreMesh(core_axis_name="c", subcore_axis_name="s")
    bpw = batch // _NW

    @functools.partial(
        pl.kernel,
        out_type=jax.ShapeDtypeStruct((batch, hist, _D), jnp.float32),
        mesh=mesh,
        compiler_params=pltpu.CompilerParams(
            needs_layout_passes=False, use_tc_tiling_on_sc=False
        ),
        scratch_types=[
            pltpu.VMEM((bpw, hist), jnp.int32),
            [pltpu.VMEM((hist, _D), jnp.float32) for _ in range(_NBUF)],
            pltpu.VMEM((_D,), jnp.float32),
            pltpu.VMEM((_D,), jnp.float32),
            [pltpu.SemaphoreType.DMA for _ in range(_NBUF)],
            [pltpu.SemaphoreType.DMA for _ in range(_NBUF)],
        ],
    )
    def kern(idx_hbm, table_hbm, gamma_hbm, beta_hbm, out_hbm,
             idx_v, rows, gamma_v, beta_v, gsem, ssem):
        wid = lax.axis_index("s") * 2 + lax.axis_index("c")
        b_base = wid * bpw
        pltpu.sync_copy(gamma_hbm, gamma_v)
        pltpu.sync_copy(beta_hbm, beta_v)
        pltpu.sync_copy(idx_hbm.at[pl.ds(b_base, bpw)], idx_v)
        gam = [gamma_v[pl.ds(16 * i, 16)] for i in range(_D // 16)]
        bet = [beta_v[pl.ds(16 * i, 16)] for i in range(_D // 16)]

        def gather(c, b):
            return pltpu.make_async_copy(
                table_hbm.at[idx_v.at[c]], rows[b], gsem[b]
            )

        def scatter(c, b):
            return pltpu.make_async_copy(
                rows[b], out_hbm.at[b_base + c], ssem[b]
            )

        def compute(b):
            @plsc.parallel_loop(0, hist, unroll=4)
            def row_body(r):
                vs = [rows[b][r, pl.ds(16 * i, 16)] for i in range(_D // 16)]
                s = jnp.sum(vs[0] + vs[1] + vs[2] + vs[3])
                q = jnp.sum(
                    vs[0] * vs[0] + vs[1] * vs[1] + vs[2] * vs[2] + vs[3] * vs[3]
                )
                mean = s * (1.0 / _D)
                var = q * (1.0 / _D) - mean * mean
                var_v = jnp.full((16,), var, jnp.float32) + _EPS
                rstd = _rsqrt(var_v)
                mrs = jnp.full((16,), mean, jnp.float32) * rstd
                for i in range(_D // 16):
                    o = (vs[i] * rstd - mrs) * gam[i] + bet[i]
                    rows[b][r, pl.ds(16 * i, 16)] = o

        for c0 in range(_LOOK):
            gather(c0, c0).start()

        def ring_body(cc, carry):
            for b in range(_NBUF):
                c = cc * _NBUF + b
                nb = (b + _LOOK) % _NBUF

                @pl.when(c + _LOOK < bpw)
                def _():
                    @pl.when(c >= _NBUF - _LOOK)
                    def _():
                        scatter(c - (_NBUF - _LOOK), nb).wait()

                    gather(c + _LOOK, nb).start()

                gather(c, b).wait()
                compute(b)
                scatter(c, b).start()
            return carry

        lax.fori_loop(0, bpw // _NBUF, ring_body, 0)
        for b in range(_NBUF):
            scatter(bpw - _NBUF + b, b).wait()

    return kern


@jax.jit
def kernel(place_ids, table, gamma, beta):
    batch, hist = place_ids.shape
    return _make_kernel(batch, hist)(
        place_ids.astype(jnp.int32), table, gamma, beta
    )
